# Initial kernel scaffold; baseline (speedup 1.0000x reference)
import jax
import jax.numpy as jnp
from jax.experimental import pallas as pl


def kernel(x, table):
    raise NotImplementedError("write your pallas kernel here")


# speedup vs baseline: 5.0343x; 5.0343x over previous
import functools

import jax
import jax.numpy as jnp
from jax import lax
from jax.experimental import pallas as pl
from jax.experimental.pallas import tpu as pltpu
from jax.experimental.pallas import tpu_sc as plsc

VOCAB = 1440
EMBED = 48
NC = 2
NS = 16
NW = NC * NS
LANE = 16
IBW = 128
NB = 4
BATCH = NB * IBW
K8 = EMBED // 8


@functools.lru_cache(maxsize=None)
def _build(n_i: int, n_j: int):
    n_batches = (n_i // IBW // NB) * n_j
    per_w = n_batches // NW
    assert per_w % 2 == 0
    bat_per_j = n_i // BATCH
    mesh = plsc.VectorSubcoreMesh(
        core_axis_name="c", subcore_axis_name="s",
        num_cores=NC, num_subcores=NS)

    @functools.partial(
        pl.kernel,
        out_type=jax.ShapeDtypeStruct((n_j, K8, n_i // IBW, 8 * IBW),
                                      jnp.float32),
        mesh=mesh,
        scratch_types=[
            pltpu.VMEM((VOCAB * EMBED,), jnp.float32),
            pltpu.VMEM((2, BATCH), jnp.int32),
            pltpu.VMEM((2, NB, K8, 8 * IBW), jnp.float32),
            pltpu.SemaphoreType.DMA,
            pltpu.SemaphoreType.DMA,
            pltpu.SemaphoreType.DMA,
            pltpu.SemaphoreType.DMA,
        ],
        compiler_params=pltpu.CompilerParams(
            use_tc_tiling_on_sc=False, needs_layout_passes=False),
    )
    def gather_kernel(table_hbm, xt_hbm, out_hbm, table_v, idx_v, trans_v,
                      si0, si1, so0, so1):
        sem_i = (si0, si1)
        sem_o = (so0, so1)
        wid = lax.axis_index("s") * NC + lax.axis_index("c")
        b0 = wid * per_w

        pltpu.sync_copy(table_hbm, table_v)

        def start_idx(b, slot):
            j = b // bat_per_j
            col = (b % bat_per_j) * BATCH
            pltpu.async_copy(
                xt_hbm.at[j, pl.ds(pl.multiple_of(col, 8), BATCH)],
                idx_v.at[slot], sem_i[slot])

        def wait_idx(slot):
            pltpu.make_async_copy(
                xt_hbm.at[0, pl.ds(0, BATCH)], idx_v.at[slot],
                sem_i[slot]).wait()

        def wait_out(slot):
            for k8 in range(K8):
                pltpu.make_async_copy(
                    trans_v.at[slot, :, k8],
                    out_hbm.at[0, k8, pl.ds(0, NB)],
                    sem_o[slot]).wait()

        start_idx(b0, 0)
        start_idx(b0 + 1, 1)

        @pl.loop(0, per_w, step=2)
        def _batches(g):
            for s in range(2):
                b = b0 + g + s
                j = b // bat_per_j
                ib0 = (b % bat_per_j) * NB

                @pl.when(g >= 2)
                def _():
                    wait_out(s)

                wait_idx(s)
                for ib in range(NB):
                    @pl.loop(0, IBW // LANE)
                    def _groups(gg):
                        off = ib * IBW + gg * LANE
                        idxv = idx_v[s, pl.ds(off, LANE)]
                        addr = idxv * EMBED
                        for k in range(EMBED):
                            val = plsc.load_gather(table_v, [addr + k])
                            trans_v[s, ib, k // 8,
                                    pl.ds((k % 8) * IBW + gg * LANE,
                                          LANE)] = val

                for k8 in range(K8):
                    pltpu.async_copy(
                        trans_v.at[s, :, k8],
                        out_hbm.at[j, k8,
                                   pl.ds(pl.multiple_of(ib0, 4), NB)],
                        sem_o[s])

                @pl.when(g < per_w - 2)
                def _():
                    start_idx(b + 2, s)

        for s in range(2):
            wait_out(s)

    return gather_kernel


def kernel(x, table):
    n_i, n_j = x.shape
    xt = x.T.astype(jnp.int32)
    tab = table.astype(jnp.float32).reshape(-1)
    out5 = _build(n_i, n_j)(tab, xt)
    out = (out5.reshape(n_j, K8, n_i // IBW, 8, IBW)
           .transpose(2, 4, 0, 1, 3)
           .reshape(n_i, n_j, EMBED))
    return out


# speedup vs baseline: 17.3316x; 3.4427x over previous
import functools

import jax
import jax.numpy as jnp
from jax import lax
from jax.experimental import pallas as pl
from jax.experimental.pallas import tpu as pltpu
from jax.experimental.pallas import tpu_sc as plsc

VOCAB = 1440
EMBED = 48
NC = 2
NS = 16
NW = NC * NS
LANE = 16
IBW = 128
NB = 4
BATCH = NB * IBW
K8 = EMBED // 8
TPAD = EMBED + 1


@functools.lru_cache(maxsize=None)
def _build(n_i: int, n_j: int):
    n_batches = (n_i // IBW // NB) * n_j
    per_w = n_batches // NW
    assert per_w % 2 == 0
    bat_per_j = n_i // BATCH
    mesh = plsc.VectorSubcoreMesh(
        core_axis_name="c", subcore_axis_name="s",
        num_cores=NC, num_subcores=NS)

    @functools.partial(
        pl.kernel,
        out_type=jax.ShapeDtypeStruct((n_j, K8, n_i // IBW, 8 * IBW),
                                      jnp.float32),
        mesh=mesh,
        scratch_types=[
            pltpu.VMEM((VOCAB * TPAD,), jnp.float32),
            pltpu.VMEM((2, BATCH), jnp.int32),
            pltpu.VMEM((2, NB, K8, 8 * IBW), jnp.float32),
            pltpu.SemaphoreType.DMA,
            pltpu.SemaphoreType.DMA,
            pltpu.SemaphoreType.DMA,
            pltpu.SemaphoreType.DMA,
        ],
        compiler_params=pltpu.CompilerParams(
            use_tc_tiling_on_sc=False, needs_layout_passes=False),
    )
    def gather_kernel(table_hbm, xt_hbm, out_hbm, table_v, idx_v, trans_v,
                      si0, si1, so0, so1):
        sem_i = (si0, si1)
        sem_o = (so0, so1)
        wid = lax.axis_index("s") * NC + lax.axis_index("c")
        b0 = wid * per_w

        pltpu.sync_copy(table_hbm, table_v)

        def start_idx(b, slot):
            j = b // bat_per_j
            col = (b % bat_per_j) * BATCH
            pltpu.async_copy(
                xt_hbm.at[j, pl.ds(pl.multiple_of(col, 8), BATCH)],
                idx_v.at[slot], sem_i[slot])

        def wait_idx(slot):
            pltpu.make_async_copy(
                xt_hbm.at[0, pl.ds(0, BATCH)], idx_v.at[slot],
                sem_i[slot]).wait()

        def wait_out(slot):
            for k8 in range(K8):
                pltpu.make_async_copy(
                    trans_v.at[slot, :, k8],
                    out_hbm.at[0, k8, pl.ds(0, NB)],
                    sem_o[slot]).wait()

        start_idx(b0, 0)
        start_idx(b0 + 1, 1)

        @pl.loop(0, per_w, step=2)
        def _batches(g):
            for s in range(2):
                b = b0 + g + s
                j = b // bat_per_j
                ib0 = (b % bat_per_j) * NB

                @pl.when(g >= 2)
                def _():
                    wait_out(s)

                wait_idx(s)
                for ib in range(NB):
                    @plsc.parallel_loop(0, IBW // LANE)
                    def _groups(gg):
                        off = ib * IBW + gg * LANE
                        idxv = idx_v[s, pl.ds(off, LANE)]
                        addr = idxv * TPAD
                        for k in range(EMBED):
                            val = plsc.load_gather(table_v, [addr + k])
                            trans_v[s, ib, k // 8,
                                    pl.ds((k % 8) * IBW + gg * LANE,
                                          LANE)] = val

                for k8 in range(K8):
                    pltpu.async_copy(
                        trans_v.at[s, :, k8],
                        out_hbm.at[j, k8,
                                   pl.ds(pl.multiple_of(ib0, 4), NB)],
                        sem_o[s])

                @pl.when(g < per_w - 2)
                def _():
                    start_idx(b + 2, s)

        for s in range(2):
            wait_out(s)

    return gather_kernel


def kernel(x, table):
    n_i, n_j = x.shape
    xt = x.T.astype(jnp.int32)
    tab = jnp.pad(table.astype(jnp.float32),
                  ((0, 0), (0, TPAD - EMBED))).reshape(-1)
    out5 = _build(n_i, n_j)(tab, xt)
    out = (out5.reshape(n_j, K8, n_i // IBW, 8, IBW)
           .transpose(2, 4, 0, 1, 3)
           .reshape(n_i, n_j, EMBED))
    return out
